# baseline (speedup 1.0000x reference)
import functools
import math

import jax
import jax.numpy as jnp
from jax.experimental import pallas as pl
from jax.experimental.pallas import tpu as pltpu

D = 256
H = 8
HID = 1024
L = 4
DH = D // H


def _dgt(a, w):
    return jax.lax.dot_general(a, w, (((1,), (1,)), ((), ())),
                               preferred_element_type=jnp.float32)


def _body(x_ref, mem_hbm, fl_ref, ipw_ref, ipb_ref, opw_ref, opb_ref,
          f1w_ref, f1b_ref, f2w_ref, f2b_ref, spw_ref, spb_ref,
          g1_ref, gb1_ref, g2_ref, gb2_ref, e_ref, out_hbm,
          m_s, o_s, in_sems, out_sems):
    f32 = jnp.float32
    t = x_ref.shape[0]
    i = pl.program_id(0)
    ng = pl.num_programs(0)
    slot = jax.lax.rem(i, 2)
    nslot = jax.lax.rem(i + 1, 2)

    def in_copy(step, sl):
        return pltpu.make_async_copy(
            mem_hbm.at[pl.ds(step * t, t)], m_s.at[sl], in_sems.at[sl])

    def out_copy(step, sl):
        return pltpu.make_async_copy(
            o_s.at[sl], out_hbm.at[pl.ds(step * t, t)], out_sems.at[sl])

    @pl.when(i == 0)
    def _():
        in_copy(i, slot).start()

    @pl.when(i + 1 < ng)
    def _():
        in_copy(i + 1, nslot).start()

    in_copy(i, slot).wait()

    @pl.when(i >= 2)
    def _():
        out_copy(i, slot).wait()

    x = x_ref[...]
    scale = 1.0 / math.sqrt(DH)

    fl = jnp.transpose(fl_ref[0])

    wq = ipw_ref[0:D, :]
    wk = ipw_ref[D:2 * D, :]
    wv = ipw_ref[2 * D:3 * D, :]
    bq = ipb_ref[:, 0:D]
    bk = ipb_ref[:, D:2 * D]
    bv = ipb_ref[:, 2 * D:3 * D]

    q = _dgt(x, wq) + bq
    m = [m_s[slot, :, l, :] for l in range(L)]
    k = [_dgt(m[l], wk) + bk for l in range(L)]
    v = [_dgt(m[l], wv) + bv for l in range(L)]

    e_exp = e_ref[...]
    s = [_dgt(q * k[l], e_exp) * scale + fl[:, l:l + 1] for l in range(L)]
    mx = jnp.maximum(jnp.maximum(s[0], s[1]), jnp.maximum(s[2], s[3]))
    ex = [jnp.exp(s[l] - mx) for l in range(L)]
    den = ex[0] + ex[1] + ex[2] + ex[3]
    a = [ex[l] / den for l in range(L)]

    o = jnp.zeros_like(x)
    for l in range(L):
        o = o + jnp.dot(a[l], e_exp, preferred_element_type=f32) * v[l]
    o = _dgt(o, opw_ref[...]) + opb_ref[...]

    def ln(y, g, b):
        mu = jnp.mean(y, axis=-1, keepdims=True)
        yc = y - mu
        var = jnp.mean(yc * yc, axis=-1, keepdims=True)
        return yc * jax.lax.rsqrt(var + 1e-5) * g + b

    e1 = ln(x + o, g1_ref[...], gb1_ref[...])
    h1 = jnp.maximum(_dgt(e1, f1w_ref[...]) + f1b_ref[...], 0.0)
    e2 = _dgt(h1, f2w_ref[...]) + f2b_ref[...]
    e3 = ln(e1 + e2, g2_ref[...], gb2_ref[...])

    valid = fl[:, 4:5]
    saved = fl[:, 5:6]
    oe = jnp.where(valid > 0, e3, x)
    se = _dgt(oe, spw_ref[...]) + spb_ref[...]

    o_s[slot, :, 0, :] = oe
    for l in range(L - 1):
        o_s[slot, :, l + 1, :] = jnp.where(saved > 0, m[l + 1], m[l])
    o_s[slot, :, L, :] = jnp.where(saved > 0, se, m[L - 1])

    out_copy(i, slot).start()

    @pl.when(i == ng - 1)
    def _():
        out_copy(i, nslot).wait()
        out_copy(i, slot).wait()


@functools.partial(jax.jit, static_argnames=())
def kernel(output_embedding, scores, mem_padding_mask, save_period, mem_bank,
           save_proj_w, save_proj_b, in_proj_w, in_proj_b, out_proj_w,
           out_proj_b, fc1_w, fc1_b, fc2_w, fc2_b, ln1_g, ln1_b, ln2_g,
           ln2_b):
    f32 = jnp.float32
    n = output_embedding.shape[0]
    x = output_embedding

    t = 512 if n % 512 == 0 else n
    g = n // t
    grid = (g,)

    ma = jnp.where(mem_padding_mask, -1e9, 0.0).astype(f32)
    valid_f = (~mem_padding_mask[:, L - 1]).astype(f32)
    saved_f = ((save_period == 0) & (scores > 0.0)).astype(f32)
    fl6 = jnp.stack([ma[:, 0].reshape(g, t), ma[:, 1].reshape(g, t),
                     ma[:, 2].reshape(g, t), ma[:, 3].reshape(g, t),
                     valid_f.reshape(g, t), saved_f.reshape(g, t)],
                    axis=1)

    ipb = in_proj_b[None, :]
    opb = out_proj_b[None, :]
    f1b = fc1_b[None, :]
    f2b = fc2_b[None, :]
    spb = save_proj_b[None, :]
    g1 = ln1_g[None, :]
    gb1 = ln1_b[None, :]
    g2 = ln2_g[None, :]
    gb2 = ln2_b[None, :]

    e_exp = jnp.repeat(jnp.eye(H, dtype=f32), DH, axis=1)

    def row_spec(width):
        return pl.BlockSpec((t, width), lambda i: (i, 0))

    def const_spec(shape):
        return pl.BlockSpec(shape, lambda i: (0,) * len(shape))

    consts = [in_proj_w, ipb, out_proj_w, opb, fc1_w, f1b, fc2_w, f2b,
              save_proj_w, spb, g1, gb1, g2, gb2, e_exp]
    out = pl.pallas_call(
        _body,
        grid=grid,
        in_specs=[row_spec(D),
                  pl.BlockSpec(memory_space=pltpu.MemorySpace.HBM),
                  pl.BlockSpec((1, 6, t), lambda i: (i, 0, 0))] +
                 [const_spec(c.shape) for c in consts],
        out_specs=pl.BlockSpec(memory_space=pltpu.MemorySpace.HBM),
        out_shape=jax.ShapeDtypeStruct((n, L + 1, D), f32),
        scratch_shapes=[
            pltpu.VMEM((2, t, L, D), f32),
            pltpu.VMEM((2, t, L + 1, D), f32),
            pltpu.SemaphoreType.DMA((2,)),
            pltpu.SemaphoreType.DMA((2,)),
        ],
    )(x, mem_bank, fl6, *consts)
    return out


# speedup vs baseline: 1.6852x; 1.6852x over previous
import functools
import math

import jax
import jax.numpy as jnp
from jax.experimental import pallas as pl
from jax.experimental.pallas import tpu as pltpu

D = 256
H = 8
HID = 1024
L = 4
DH = D // H


def _dgt(a, w):
    return jax.lax.dot_general(a, w, (((1,), (1,)), ((), ())),
                               preferred_element_type=jnp.float32)


def _body(x_ref, mem_hbm, fl_ref, ipw_ref, ipb_ref, opw_ref, opb_ref,
          f1w_ref, f1b_ref, f2w_ref, f2b_ref, spw_ref, spb_ref,
          g1_ref, gb1_ref, g2_ref, gb2_ref, e_ref, out_hbm,
          m_s, o_s, in_sems, out_sems):
    f32 = jnp.float32
    t = x_ref.shape[0]
    i = pl.program_id(0)
    ng = pl.num_programs(0)
    slot = jax.lax.rem(i, 2)
    nslot = jax.lax.rem(i + 1, 2)

    def in_copy(step, sl):
        return pltpu.make_async_copy(
            mem_hbm.at[pl.ds(step * t, t)], m_s.at[sl], in_sems.at[sl])

    def out_copy(step, sl):
        return pltpu.make_async_copy(
            o_s.at[sl], out_hbm.at[:, pl.ds(step * t, t), :],
            out_sems.at[sl])

    @pl.when(i == 0)
    def _():
        in_copy(i, slot).start()

    @pl.when(i + 1 < ng)
    def _():
        in_copy(i + 1, nslot).start()

    in_copy(i, slot).wait()

    @pl.when(i >= 2)
    def _():
        out_copy(i, slot).wait()

    x = x_ref[...]
    scale = 1.0 / math.sqrt(DH)

    fl = jnp.transpose(fl_ref[0])

    wq = ipw_ref[0:D, :]
    wk = ipw_ref[D:2 * D, :]
    wv = ipw_ref[2 * D:3 * D, :]
    bq = ipb_ref[:, 0:D]
    bk = ipb_ref[:, D:2 * D]
    bv = ipb_ref[:, 2 * D:3 * D]

    q = _dgt(x, wq) + bq
    m = [m_s[slot, :, l, :] for l in range(L)]
    k = [_dgt(m[l], wk) + bk for l in range(L)]
    v = [_dgt(m[l], wv) + bv for l in range(L)]

    e_exp = e_ref[...]
    s = [_dgt(q * k[l], e_exp) * scale + fl[:, l:l + 1] for l in range(L)]
    mx = jnp.maximum(jnp.maximum(s[0], s[1]), jnp.maximum(s[2], s[3]))
    ex = [jnp.exp(s[l] - mx) for l in range(L)]
    den = ex[0] + ex[1] + ex[2] + ex[3]
    a = [ex[l] / den for l in range(L)]

    o = jnp.zeros_like(x)
    for l in range(L):
        o = o + jnp.dot(a[l], e_exp, preferred_element_type=f32) * v[l]
    o = _dgt(o, opw_ref[...]) + opb_ref[...]

    def ln(y, g, b):
        mu = jnp.mean(y, axis=-1, keepdims=True)
        yc = y - mu
        var = jnp.mean(yc * yc, axis=-1, keepdims=True)
        return yc * jax.lax.rsqrt(var + 1e-5) * g + b

    e1 = ln(x + o, g1_ref[...], gb1_ref[...])
    h1 = jnp.maximum(_dgt(e1, f1w_ref[...]) + f1b_ref[...], 0.0)
    e2 = _dgt(h1, f2w_ref[...]) + f2b_ref[...]
    e3 = ln(e1 + e2, g2_ref[...], gb2_ref[...])

    valid = fl[:, 4:5]
    saved = fl[:, 5:6]
    oe = jnp.where(valid > 0, e3, x)
    se = _dgt(oe, spw_ref[...]) + spb_ref[...]

    o_s[slot, 0] = oe
    for l in range(L - 1):
        o_s[slot, l + 1] = jnp.where(saved > 0, m[l + 1], m[l])
    o_s[slot, L] = jnp.where(saved > 0, se, m[L - 1])

    out_copy(i, slot).start()

    @pl.when(i == ng - 1)
    def _():
        out_copy(i, nslot).wait()
        out_copy(i, slot).wait()


@functools.partial(jax.jit, static_argnames=())
def kernel(output_embedding, scores, mem_padding_mask, save_period, mem_bank,
           save_proj_w, save_proj_b, in_proj_w, in_proj_b, out_proj_w,
           out_proj_b, fc1_w, fc1_b, fc2_w, fc2_b, ln1_g, ln1_b, ln2_g,
           ln2_b):
    f32 = jnp.float32
    n = output_embedding.shape[0]
    x = output_embedding

    t = 512 if n % 512 == 0 else n
    g = n // t
    grid = (g,)

    ma = jnp.where(mem_padding_mask, -1e9, 0.0).astype(f32)
    valid_f = (~mem_padding_mask[:, L - 1]).astype(f32)
    saved_f = ((save_period == 0) & (scores > 0.0)).astype(f32)
    fl6 = jnp.stack([ma[:, 0].reshape(g, t), ma[:, 1].reshape(g, t),
                     ma[:, 2].reshape(g, t), ma[:, 3].reshape(g, t),
                     valid_f.reshape(g, t), saved_f.reshape(g, t)],
                    axis=1)

    ipb = in_proj_b[None, :]
    opb = out_proj_b[None, :]
    f1b = fc1_b[None, :]
    f2b = fc2_b[None, :]
    spb = save_proj_b[None, :]
    g1 = ln1_g[None, :]
    gb1 = ln1_b[None, :]
    g2 = ln2_g[None, :]
    gb2 = ln2_b[None, :]

    e_exp = jnp.repeat(jnp.eye(H, dtype=f32), DH, axis=1)

    def row_spec(width):
        return pl.BlockSpec((t, width), lambda i: (i, 0))

    def const_spec(shape):
        return pl.BlockSpec(shape, lambda i: (0,) * len(shape))

    consts = [in_proj_w, ipb, out_proj_w, opb, fc1_w, f1b, fc2_w, f2b,
              save_proj_w, spb, g1, gb1, g2, gb2, e_exp]
    out = pl.pallas_call(
        _body,
        grid=grid,
        in_specs=[row_spec(D),
                  pl.BlockSpec(memory_space=pltpu.MemorySpace.HBM),
                  pl.BlockSpec((1, 6, t), lambda i: (i, 0, 0))] +
                 [const_spec(c.shape) for c in consts],
        out_specs=pl.BlockSpec(memory_space=pltpu.MemorySpace.HBM),
        out_shape=jax.ShapeDtypeStruct((L + 1, n, D), f32),
        scratch_shapes=[
            pltpu.VMEM((2, t, L, D), f32),
            pltpu.VMEM((2, L + 1, t, D), f32),
            pltpu.SemaphoreType.DMA((2,)),
            pltpu.SemaphoreType.DMA((2,)),
        ],
    )(x, mem_bank, fl6, *consts)
    return jnp.transpose(out, (1, 0, 2))


# speedup vs baseline: 1.9354x; 1.1485x over previous
import functools
import math

import jax
import jax.numpy as jnp
from jax.experimental import pallas as pl
from jax.experimental.pallas import tpu as pltpu

D = 256
H = 8
HID = 1024
L = 4
DH = D // H


def _dgt(a, w):
    return jax.lax.dot_general(a, w, (((1,), (1,)), ((), ())),
                               preferred_element_type=jnp.float32)


def _body(x_ref, mem_hbm, fl_ref, ipw_ref, ipb_ref, opw_ref, opb_ref,
          f1w_ref, f1b_ref, f2w_ref, f2b_ref, spw_ref, spb_ref,
          g1_ref, gb1_ref, g2_ref, gb2_ref, e_ref, out_hbm,
          m_s, o_s, in_sems, out_sems):
    f32 = jnp.float32
    t = x_ref.shape[0]
    i = pl.program_id(0)
    ng = pl.num_programs(0)
    slot = jax.lax.rem(i, 2)
    nslot = jax.lax.rem(i + 1, 2)

    def in_copy(step, sl):
        return pltpu.make_async_copy(
            mem_hbm.at[pl.ds(step * t, t)], m_s.at[sl], in_sems.at[sl])

    def out_copy(step, sl):
        return pltpu.make_async_copy(
            o_s.at[sl], out_hbm.at[:, pl.ds(step * t, t), :],
            out_sems.at[sl])

    @pl.when(i == 0)
    def _():
        in_copy(i, slot).start()

    @pl.when(i + 1 < ng)
    def _():
        in_copy(i + 1, nslot).start()

    in_copy(i, slot).wait()

    @pl.when(i >= 2)
    def _():
        out_copy(i, slot).wait()

    x = x_ref[...]
    scale = 1.0 / math.sqrt(DH)

    fl = jnp.transpose(fl_ref[0])

    wq = ipw_ref[0:D, :]
    wk = ipw_ref[D:2 * D, :]
    wv = ipw_ref[2 * D:3 * D, :]
    bq = ipb_ref[:, 0:D]
    bk = ipb_ref[:, D:2 * D]
    bv = ipb_ref[:, 2 * D:3 * D]

    q = _dgt(x, wq) + bq
    m = [m_s[slot, :, l, :] for l in range(L)]
    k = [_dgt(m[l], wk) + bk for l in range(L)]
    v = [_dgt(m[l], wv) + bv for l in range(L)]

    e_exp = e_ref[...]
    s = [_dgt(q * k[l], e_exp) * scale + fl[:, l:l + 1] for l in range(L)]
    mx = jnp.maximum(jnp.maximum(s[0], s[1]), jnp.maximum(s[2], s[3]))
    ex = [jnp.exp(s[l] - mx) for l in range(L)]
    den = ex[0] + ex[1] + ex[2] + ex[3]
    a = [ex[l] / den for l in range(L)]

    o = jnp.zeros_like(x)
    for l in range(L):
        o = o + jnp.dot(a[l], e_exp, preferred_element_type=f32) * v[l]
    o = _dgt(o, opw_ref[...]) + opb_ref[...]

    def ln(y, g, b):
        mu = jnp.mean(y, axis=-1, keepdims=True)
        yc = y - mu
        var = jnp.mean(yc * yc, axis=-1, keepdims=True)
        return yc * jax.lax.rsqrt(var + 1e-5) * g + b

    e1 = ln(x + o, g1_ref[...], gb1_ref[...])
    h1 = jnp.maximum(_dgt(e1, f1w_ref[...]) + f1b_ref[...], 0.0)
    e2 = _dgt(h1, f2w_ref[...]) + f2b_ref[...]
    e3 = ln(e1 + e2, g2_ref[...], gb2_ref[...])

    valid = fl[:, 4:5]
    saved = fl[:, 5:6]
    oe = jnp.where(valid > 0, e3, x)
    se = _dgt(oe, spw_ref[...]) + spb_ref[...]

    o_s[slot, 0] = oe
    for l in range(L - 1):
        o_s[slot, l + 1] = jnp.where(saved > 0, m[l + 1], m[l])
    o_s[slot, L] = jnp.where(saved > 0, se, m[L - 1])

    out_copy(i, slot).start()

    @pl.when(i == ng - 1)
    def _():
        out_copy(i, nslot).wait()
        out_copy(i, slot).wait()


@functools.partial(jax.jit, static_argnames=())
def kernel(output_embedding, scores, mem_padding_mask, save_period, mem_bank,
           save_proj_w, save_proj_b, in_proj_w, in_proj_b, out_proj_w,
           out_proj_b, fc1_w, fc1_b, fc2_w, fc2_b, ln1_g, ln1_b, ln2_g,
           ln2_b):
    f32 = jnp.float32
    n = output_embedding.shape[0]
    x = output_embedding

    t = 1024 if n % 1024 == 0 else n
    g = n // t
    grid = (g,)

    ma = jnp.where(mem_padding_mask, -1e9, 0.0).astype(f32)
    valid_f = (~mem_padding_mask[:, L - 1]).astype(f32)
    saved_f = ((save_period == 0) & (scores > 0.0)).astype(f32)
    fl6 = jnp.stack([ma[:, 0].reshape(g, t), ma[:, 1].reshape(g, t),
                     ma[:, 2].reshape(g, t), ma[:, 3].reshape(g, t),
                     valid_f.reshape(g, t), saved_f.reshape(g, t)],
                    axis=1)

    ipb = in_proj_b[None, :]
    opb = out_proj_b[None, :]
    f1b = fc1_b[None, :]
    f2b = fc2_b[None, :]
    spb = save_proj_b[None, :]
    g1 = ln1_g[None, :]
    gb1 = ln1_b[None, :]
    g2 = ln2_g[None, :]
    gb2 = ln2_b[None, :]

    e_exp = jnp.repeat(jnp.eye(H, dtype=f32), DH, axis=1)

    def row_spec(width):
        return pl.BlockSpec((t, width), lambda i: (i, 0))

    def const_spec(shape):
        return pl.BlockSpec(shape, lambda i: (0,) * len(shape))

    consts = [in_proj_w, ipb, out_proj_w, opb, fc1_w, f1b, fc2_w, f2b,
              save_proj_w, spb, g1, gb1, g2, gb2, e_exp]
    out = pl.pallas_call(
        _body,
        grid=grid,
        in_specs=[row_spec(D),
                  pl.BlockSpec(memory_space=pltpu.MemorySpace.HBM),
                  pl.BlockSpec((1, 6, t), lambda i: (i, 0, 0))] +
                 [const_spec(c.shape) for c in consts],
        out_specs=pl.BlockSpec(memory_space=pltpu.MemorySpace.HBM),
        out_shape=jax.ShapeDtypeStruct((L + 1, n, D), f32),
        scratch_shapes=[
            pltpu.VMEM((2, t, L, D), f32),
            pltpu.VMEM((2, L + 1, t, D), f32),
            pltpu.SemaphoreType.DMA((2,)),
            pltpu.SemaphoreType.DMA((2,)),
        ],
    )(x, mem_bank, fl6, *consts)
    return jnp.transpose(out, (1, 0, 2))
